# baseline (speedup 1.0000x reference)
import jax
import jax.numpy as jnp
from jax.experimental import pallas as pl

NID = 3
EMB = 128

_BN_SCALE = 1.0 / jnp.sqrt(1.0 + 1e-5)


def _parser(params, x):
    h = jax.nn.relu(x @ params["pewe_l1_w"] + params["pewe_l1_b"])
    h = params["pewe_bn_g"] * h * _BN_SCALE + params["pewe_bn_b"]
    return h @ params["pewe_l2_w"] + params["pewe_l2_b"]


def _L(x, src, dst, norm, n):
    return jax.ops.segment_sum(norm[:, None] * x[src], dst, num_segments=n)


def kernel(image_features, edge_index, non_image_features, params):
    with jax.default_matmul_precision("float32"):
        return _impl(image_features, edge_index, non_image_features, params)


def _impl(image_features, edge_index, non_image_features, params):
    src = edge_index[0]
    dst = edge_index[1]
    n = image_features.shape[0]

    ei = image_features @ params["EI_w"] + params["EI_b"]
    es = image_features @ params["ES_w"] + params["ES_b"]
    sn = jnp.sum(es * es, axis=1)
    recon = jnp.concatenate([ei, es], axis=1) @ params["DE_w"] + params["DE_b"]

    p1 = _parser(params, non_image_features[:, :NID])
    p2 = _parser(params, non_image_features[:, NID:])
    pd = jnp.sum(p1 * p2, axis=1)
    q1 = jnp.sum(p1 * p1, axis=1)
    q2 = jnp.sum(p2 * p2, axis=1)
    ed = jnp.sum(es[src] * es[dst], axis=1)
    num = pd + ed
    den2 = jnp.maximum((q1 + sn[src]) * (q2 + sn[dst]), 1e-16)
    ew = (num * jax.lax.rsqrt(den2) + 1.0) * 0.5

    deg = jax.ops.segment_sum(ew, src, num_segments=n)
    dinv = jnp.where(deg > 0, jax.lax.rsqrt(jnp.where(deg > 0, deg, 1.0)), 0.0)
    norm = -ew * dinv[src] * dinv[dst]

    W = params["cheb0"]
    c0a = ei @ (W[0] - W[2])
    u12 = jnp.concatenate([ei @ W[1], ei @ W[2]], axis=1)
    P = _L(u12, src, dst, norm, n)
    t = P[:, 16:]
    h = jax.nn.relu(c0a + P[:, :16] + 2.0 * _L(t, src, dst, norm, n))

    hs = [h]
    for name in ("cheb1", "cheb2", "cheb3"):
        W = params[name]
        Tx1 = _L(h, src, dst, norm, n)
        Tx2 = 2.0 * _L(Tx1, src, dst, norm, n) - h
        h = jax.nn.relu(h @ W[0] + Tx1 @ W[1] + Tx2 @ W[2])
        hs.append(h)

    jk = jnp.concatenate(hs, axis=1)
    z = jax.nn.relu(jk @ params["lab1_w"] + params["lab1_b"])
    z = params["lab_bn_g"] * z * _BN_SCALE + params["lab_bn_b"]
    label_logits = z @ params["lab2_w"] + params["lab2_b"]

    s = jax.nn.relu(ei @ params["site1_w"] + params["site1_b"])
    s = params["site_bn_g"] * s * _BN_SCALE + params["site_bn_b"]
    site_logits = s @ params["site2_w"] + params["site2_b"]

    return (label_logits, site_logits, es, recon)


# speedup vs baseline: 6.2581x; 6.2581x over previous
import functools

import jax
import jax.numpy as jnp
from jax import lax
from jax.experimental import pallas as pl
from jax.experimental.pallas import tpu as pltpu
from jax.experimental.pallas import tpu_sc as plsc

N = 10000
E = 320000
NP = 10240
NC, NS = 2, 16
NW = NC * NS
EPT = E // NW
CH = 80
NCH = EPT // CH
SPAN = NP // NS
BN_S = float(1.0 / (1.0 + 1e-5) ** 0.5)

_f32 = jnp.float32


def _rsqrt16(x):
    i = plsc.bitcast(x, jnp.int32)
    i = jnp.full((16,), 0x5F3759DF, jnp.int32) - lax.shift_right_logical(i, 1)
    y = plsc.bitcast(i, _f32)
    for _ in range(4):
        y = y * (1.5 - 0.5 * x * y * y)
    return y


def _zero16():
    return jnp.zeros((16,), _f32)


def _sc_ew_deg(es, src, dst, pd, q1, q2, sn):
    mesh = plsc.VectorSubcoreMesh(core_axis_name="c", subcore_axis_name="s")

    @functools.partial(
        pl.kernel,
        out_type=(jax.ShapeDtypeStruct((E,), _f32),
                  jax.ShapeDtypeStruct((NC, NP), _f32)),
        mesh=mesh,
        compiler_params=pltpu.CompilerParams(needs_layout_passes=False, use_tc_tiling_on_sc=False),
        scratch_types=[
            pltpu.VMEM((N,), _f32),
            pltpu.VMEM((CH,), jnp.int32),
            pltpu.VMEM((CH,), jnp.int32),
            pltpu.VMEM((CH,), _f32),
            pltpu.VMEM((CH,), _f32),
            pltpu.VMEM((CH,), _f32),
            pltpu.VMEM((CH, 128), _f32),
            pltpu.VMEM((CH, 128), _f32),
            pltpu.VMEM((CH * 16,), _f32),
            pltpu.VMEM((CH,), _f32),
            pltpu.VMEM((SPAN,), _f32),
            pltpu.VMEM_SHARED((NP,), _f32),
            pltpu.SemaphoreType.DMA,
        ],
    )
    def k(es_h, src_h, dst_h, pd_h, q1_h, q2_h, sn_h, ew_h, deg_h,
          sn_v, idx_s, idx_d, pd_v, q1_v, q2_v, e1_v, e2_v, prs, ew_v,
          zv, deg_sp, sem):
        cid = lax.axis_index("c")
        sid = lax.axis_index("s")
        base = (sid * NC + cid) * EPT

        def zb(i, c):
            zv[pl.ds(i * 16, 16)] = _zero16()
            return c
        lax.fori_loop(0, SPAN // 16, zb, 0)
        pltpu.sync_copy(zv, deg_sp.at[pl.ds(sid * SPAN, SPAN)])
        pltpu.sync_copy(sn_h, sn_v)
        plsc.subcore_barrier()

        iota = lax.iota(jnp.int32, 16)

        def chunk(kk, c):
            off = base + kk * CH
            pltpu.sync_copy(src_h.at[pl.ds(off, CH)], idx_s)
            pltpu.sync_copy(dst_h.at[pl.ds(off, CH)], idx_d)
            pltpu.sync_copy(pd_h.at[pl.ds(off, CH)], pd_v)
            pltpu.sync_copy(q1_h.at[pl.ds(off, CH)], q1_v)
            pltpu.sync_copy(q2_h.at[pl.ds(off, CH)], q2_v)
            g1 = pltpu.async_copy(es_h.at[idx_s], e1_v, sem)
            g2 = pltpu.async_copy(es_h.at[idx_d], e2_v, sem)
            g1.wait()
            g2.wait()

            def edot(e, cc):
                acc = e1_v[e, pl.ds(0, 16)] * e2_v[e, pl.ds(0, 16)]
                for t in range(1, 8):
                    acc = acc + (e1_v[e, pl.ds(t * 16, 16)]
                                 * e2_v[e, pl.ds(t * 16, 16)])
                prs[pl.ds(e * 16, 16)] = acc
                return cc
            lax.fori_loop(0, CH, edot, 0)

            def group(g, cc):
                rows = (g * 16 + iota) * 16
                ed = plsc.load_gather(prs, [rows])
                for j in range(1, 16):
                    ed = ed + plsc.load_gather(prs, [rows + j])
                s16 = idx_s[pl.ds(g * 16, 16)]
                d16 = idx_d[pl.ds(g * 16, 16)]
                sns = plsc.load_gather(sn_v, [s16])
                snd = plsc.load_gather(sn_v, [d16])
                num = pd_v[pl.ds(g * 16, 16)] + ed
                den2 = jnp.maximum(
                    (q1_v[pl.ds(g * 16, 16)] + sns)
                    * (q2_v[pl.ds(g * 16, 16)] + snd), 1e-16)
                ew_v[pl.ds(g * 16, 16)] = (num * _rsqrt16(den2) + 1.0) * 0.5
                return cc
            lax.fori_loop(0, CH // 16, group, 0)

            pltpu.sync_copy(ew_v, ew_h.at[pl.ds(off, CH)])
            pltpu.sync_copy(ew_v, deg_sp.at[idx_s], add=True)
            return c
        lax.fori_loop(0, NCH, chunk, 0)

        plsc.subcore_barrier()
        pltpu.sync_copy(deg_sp.at[pl.ds(sid * SPAN, SPAN)],
                        deg_h.at[cid, pl.ds(sid * SPAN, SPAN)])

    return k(es, src, dst, pd, q1, q2, sn)


def _sc_l_apply(x, src, dst, F, norm=None, ew=None, deg=None):
    mesh = plsc.VectorSubcoreMesh(core_axis_name="c", subcore_axis_name="s")
    with_norm = norm is None
    out_type = [jax.ShapeDtypeStruct((NC, NP, F), _f32)]
    if with_norm:
        out_type.append(jax.ShapeDtypeStruct((E,), _f32))
    scratch = [
        pltpu.VMEM((NP,), _f32),
        pltpu.VMEM((NP,), _f32),
        pltpu.VMEM((CH,), jnp.int32),
        pltpu.VMEM((CH,), jnp.int32),
        pltpu.VMEM((CH,), _f32),
        pltpu.VMEM((CH + 16,), _f32),
        pltpu.VMEM((CH, F), _f32),
        pltpu.VMEM((SPAN, F), _f32),
        pltpu.VMEM_SHARED((NP, F), _f32),
        pltpu.SemaphoreType.DMA,
    ]

    def body(*refs):
        if with_norm:
            (x_h, src_h, dst_h, ew_h, deg_h, acc_h, norm_h,
             dinv_v, tmp_v, idx_s, idx_d, ew_v, nm_v, rows_v, zv,
             acc_sp, sem) = refs
        else:
            (x_h, src_h, dst_h, norm_h, acc_h,
             dinv_v, tmp_v, idx_s, idx_d, ew_v, nm_v, rows_v, zv,
             acc_sp, sem) = refs
        cid = lax.axis_index("c")
        sid = lax.axis_index("s")
        base = (sid * NC + cid) * EPT
        iota = lax.iota(jnp.int32, 16)

        if with_norm:
            pltpu.sync_copy(deg_h.at[0], dinv_v)
            pltpu.sync_copy(deg_h.at[1], tmp_v)

            def db(i, c):
                d = dinv_v[pl.ds(i * 16, 16)] + tmp_v[pl.ds(i * 16, 16)]
                r = _rsqrt16(jnp.maximum(d, 1e-30))
                dinv_v[pl.ds(i * 16, 16)] = jnp.where(d > 0.0, r, 0.0)
                return c
            lax.fori_loop(0, NP // 16, db, 0)

        def zb(i, c):
            for t in range(F // 16):
                zv[i, pl.ds(t * 16, 16)] = _zero16()
            return c
        lax.fori_loop(0, SPAN, zb, 0)
        pltpu.sync_copy(zv, acc_sp.at[pl.ds(sid * SPAN, SPAN)])
        plsc.subcore_barrier()

        def chunk(kk, c):
            off = base + kk * CH
            pltpu.sync_copy(src_h.at[pl.ds(off, CH)], idx_s)
            pltpu.sync_copy(dst_h.at[pl.ds(off, CH)], idx_d)
            g = pltpu.async_copy(x_h.at[idx_s], rows_v, sem)
            if with_norm:
                pltpu.sync_copy(ew_h.at[pl.ds(off, CH)], ew_v)
            else:
                pltpu.sync_copy(norm_h.at[pl.ds(off, CH)],
                                nm_v.at[pl.ds(0, CH)])
            g.wait()

            if with_norm:
                def grp(gg, cc):
                    s16 = idx_s[pl.ds(gg * 16, 16)]
                    d16 = idx_d[pl.ds(gg * 16, 16)]
                    nv = (-ew_v[pl.ds(gg * 16, 16)]
                          * plsc.load_gather(dinv_v, [s16])
                          * plsc.load_gather(dinv_v, [d16]))
                    nm_v[pl.ds(gg * 16, 16)] = nv
                    return cc
                lax.fori_loop(0, CH // 16, grp, 0)
                pltpu.sync_copy(nm_v.at[pl.ds(0, CH)],
                                norm_h.at[pl.ds(off, CH)])

            def scale(e, cc):
                nv = jnp.full((16,), nm_v[pl.ds(e, 16)][0])
                for t in range(F // 16):
                    rows_v[e, pl.ds(t * 16, 16)] = (
                        rows_v[e, pl.ds(t * 16, 16)] * nv)
                return cc
            lax.fori_loop(0, CH, scale, 0)

            pltpu.sync_copy(rows_v, acc_sp.at[idx_d], add=True)
            return c
        lax.fori_loop(0, NCH, chunk, 0)

        plsc.subcore_barrier()
        pltpu.sync_copy(acc_sp.at[pl.ds(sid * SPAN, SPAN)],
                        acc_h.at[cid, pl.ds(sid * SPAN, SPAN)])

    kfn = functools.partial(
        pl.kernel, out_type=tuple(out_type), mesh=mesh,
        scratch_types=scratch,
        compiler_params=pltpu.CompilerParams(needs_layout_passes=False, use_tc_tiling_on_sc=False))(body)
    if with_norm:
        return kfn(x, src, dst, ew, deg)
    return kfn(x, src, dst, norm)


_BN_NODE = 2000
_GRID_N = N // _BN_NODE


def _row_spec(width):
    return pl.BlockSpec((_BN_NODE, width), lambda i: (i, 0))


def _full_spec(shape):
    nd = len(shape)
    return pl.BlockSpec(shape, lambda i: (0,) * nd)


def _tc_premix(x, p):
    eib = p["EI_b"].reshape(1, 128)
    esb = p["ES_b"].reshape(1, 128)
    deb = p["DE_b"].reshape(1, 128)
    cwa = p["cheb0"][0] - p["cheb0"][2]
    cw12 = jnp.concatenate([p["cheb0"][1], p["cheb0"][2]], axis=1)
    s1b = p["site1_b"].reshape(1, 256)
    sg = (p["site_bn_g"] * BN_S).reshape(1, 256)
    sb = p["site_bn_b"].reshape(1, 256)
    s2b = p["site2_b"].reshape(1, 20)

    def body(x_r, eiw, eib_r, esw, esb_r, dew, deb_r, cwa_r, cw12_r,
             s1w, s1b_r, sg_r, sb_r, s2w, s2b_r,
             ei_o, es_o, rec_o, sn_o, c0a_o, u12_o, site_o):
        xx = x_r[...]
        ei = jnp.dot(xx, eiw[...], preferred_element_type=_f32) + eib_r[...]
        es = jnp.dot(xx, esw[...], preferred_element_type=_f32) + esb_r[...]
        ei_o[...] = ei
        es_o[...] = es
        dw = dew[...]
        rec_o[...] = (jnp.dot(ei, dw[:128, :], preferred_element_type=_f32)
                      + jnp.dot(es, dw[128:, :], preferred_element_type=_f32)
                      + deb_r[...])
        sn_o[...] = jnp.sum(es * es, axis=1, keepdims=True)
        c0a_o[...] = jnp.dot(ei, cwa_r[...], preferred_element_type=_f32)
        u12_o[...] = jnp.dot(ei, cw12_r[...], preferred_element_type=_f32)
        s = jnp.maximum(
            jnp.dot(ei, s1w[...], preferred_element_type=_f32) + s1b_r[...],
            0.0)
        s = s * sg_r[...] + sb_r[...]
        site_o[...] = jnp.dot(s, s2w[...], preferred_element_type=_f32) + s2b_r[...]

    return pl.pallas_call(
        body,
        grid=(_GRID_N,),
        in_specs=[
            _row_spec(128),
            _full_spec((128, 128)), _full_spec((1, 128)),
            _full_spec((128, 128)), _full_spec((1, 128)),
            _full_spec((256, 128)), _full_spec((1, 128)),
            _full_spec((128, 16)), _full_spec((128, 32)),
            _full_spec((128, 256)), _full_spec((1, 256)),
            _full_spec((1, 256)), _full_spec((1, 256)),
            _full_spec((256, 20)), _full_spec((1, 20)),
        ],
        out_specs=[
            _row_spec(128), _row_spec(128), _row_spec(128), _row_spec(1),
            _row_spec(16), _row_spec(32), _row_spec(20),
        ],
        out_shape=[
            jax.ShapeDtypeStruct((N, 128), _f32),
            jax.ShapeDtypeStruct((N, 128), _f32),
            jax.ShapeDtypeStruct((N, 128), _f32),
            jax.ShapeDtypeStruct((N, 1), _f32),
            jax.ShapeDtypeStruct((N, 16), _f32),
            jax.ShapeDtypeStruct((N, 32), _f32),
            jax.ShapeDtypeStruct((N, 20), _f32),
        ],
    )(x, p["EI_w"], eib, p["ES_w"], esb, p["DE_w"], deb, cwa, cw12,
      p["site1_w"], s1b, sg, sb, p["site2_w"], s2b)


_BE = 2000
_GRID_E = E // _BE


def _tc_parser(nf, p):
    zpad = jnp.zeros((E, 5), _f32)
    nfp = jnp.concatenate([nf[:, :3], zpad, nf[:, 3:], zpad], axis=1)
    w1 = jnp.concatenate([p["pewe_l1_w"], jnp.zeros((5, 128), _f32)], axis=0)
    b1 = p["pewe_l1_b"].reshape(1, 128)
    g1 = (p["pewe_bn_g"] * BN_S).reshape(1, 128)
    bb1 = p["pewe_bn_b"].reshape(1, 128)
    b2 = p["pewe_l2_b"].reshape(1, 128)

    def body(nf_r, w1_r, b1_r, g1_r, bb1_r, w2_r, b2_r, pd_o, q1_o, q2_o):
        xf = nf_r[...]

        def pars(xx):
            h = jnp.maximum(
                jnp.dot(xx, w1_r[...], preferred_element_type=_f32)
                + b1_r[...], 0.0)
            h = h * g1_r[...] + bb1_r[...]
            return jnp.dot(h, w2_r[...], preferred_element_type=_f32) + b2_r[...]

        p1 = pars(xf[:, 0:8])
        p2 = pars(xf[:, 8:16])
        pd_o[...] = jnp.sum(p1 * p2, axis=1, keepdims=True)
        q1_o[...] = jnp.sum(p1 * p1, axis=1, keepdims=True)
        q2_o[...] = jnp.sum(p2 * p2, axis=1, keepdims=True)

    espec = pl.BlockSpec((_BE, 16), lambda i: (i, 0))
    ospec = pl.BlockSpec((_BE, 1), lambda i: (i, 0))
    return pl.pallas_call(
        body,
        grid=(_GRID_E,),
        in_specs=[espec,
                  _full_spec((8, 128)), _full_spec((1, 128)),
                  _full_spec((1, 128)), _full_spec((1, 128)),
                  _full_spec((128, 128)), _full_spec((1, 128))],
        out_specs=[ospec, ospec, ospec],
        out_shape=[jax.ShapeDtypeStruct((E, 1), _f32)] * 3,
    )(nfp, w1, b1, g1, bb1, p["pewe_l2_w"], b2)


def _tc_combine0(P, c0a):
    def body(p0_r, p1_r, c0a_r, pre_o, t_o):
        s = p0_r[...] + p1_r[...]
        pre_o[...] = c0a_r[...] + s[:, :16]
        t_o[...] = s[:, 16:]

    return pl.pallas_call(
        body,
        grid=(_GRID_N,),
        in_specs=[_row_spec(32), _row_spec(32), _row_spec(16)],
        out_specs=[_row_spec(16), _row_spec(16)],
        out_shape=[jax.ShapeDtypeStruct((N, 16), _f32)] * 2,
    )(P[0], P[1], c0a)


def _tc_h1(pre0, Q):
    def body(pre_r, q0_r, q1_r, h_o):
        h_o[...] = jnp.maximum(
            pre_r[...] + 2.0 * (q0_r[...] + q1_r[...]), 0.0)

    return pl.pallas_call(
        body,
        grid=(_GRID_N,),
        in_specs=[_row_spec(16), _row_spec(16), _row_spec(16)],
        out_specs=_row_spec(16),
        out_shape=jax.ShapeDtypeStruct((N, 16), _f32),
    )(pre0, Q[0], Q[1])


def _tc_addhalves(R):
    def body(r0_r, r1_r, o_r):
        o_r[...] = r0_r[...] + r1_r[...]

    return pl.pallas_call(
        body,
        grid=(_GRID_N,),
        in_specs=[_row_spec(16), _row_spec(16)],
        out_specs=_row_spec(16),
        out_shape=jax.ShapeDtypeStruct((N, 16), _f32),
    )(R[0], R[1])


def _tc_layer(h, tx1, S, W):
    wcat = jnp.concatenate([W[0], W[1], W[2]], axis=0)

    def body(h_r, t_r, s0_r, s1_r, w_r, o_r):
        hh = h_r[...]
        tx2 = 2.0 * (s0_r[...] + s1_r[...]) - hh
        xcat = jnp.concatenate([hh, t_r[...], tx2], axis=1)
        o_r[...] = jnp.maximum(
            jnp.dot(xcat, w_r[...], preferred_element_type=_f32), 0.0)

    return pl.pallas_call(
        body,
        grid=(_GRID_N,),
        in_specs=[_row_spec(16), _row_spec(16), _row_spec(16), _row_spec(16),
                  _full_spec((48, 16))],
        out_specs=_row_spec(16),
        out_shape=jax.ShapeDtypeStruct((N, 16), _f32),
    )(h, tx1, S[0], S[1], wcat)


def _tc_final(h1, h2, h3, tx1, S, W, p):
    wcat = jnp.concatenate([W[0], W[1], W[2]], axis=0)
    l1b = p["lab1_b"].reshape(1, 256)
    lg = (p["lab_bn_g"] * BN_S).reshape(1, 256)
    lb = p["lab_bn_b"].reshape(1, 256)
    l2b = p["lab2_b"].reshape(1, 2)

    def body(h1_r, h2_r, h3_r, t_r, s0_r, s1_r, w_r,
             l1w_r, l1b_r, lg_r, lb_r, l2w_r, l2b_r, o_r):
        h3v = h3_r[...]
        tx2 = 2.0 * (s0_r[...] + s1_r[...]) - h3v
        xcat = jnp.concatenate([h3v, t_r[...], tx2], axis=1)
        h4 = jnp.maximum(
            jnp.dot(xcat, w_r[...], preferred_element_type=_f32), 0.0)
        jk = jnp.concatenate([h1_r[...], h2_r[...], h3v, h4], axis=1)
        z = jnp.maximum(
            jnp.dot(jk, l1w_r[...], preferred_element_type=_f32)
            + l1b_r[...], 0.0)
        z = z * lg_r[...] + lb_r[...]
        o_r[...] = jnp.dot(z, l2w_r[...], preferred_element_type=_f32) + l2b_r[...]

    return pl.pallas_call(
        body,
        grid=(_GRID_N,),
        in_specs=[_row_spec(16)] * 6 + [
            _full_spec((48, 16)),
            _full_spec((64, 256)), _full_spec((1, 256)),
            _full_spec((1, 256)), _full_spec((1, 256)),
            _full_spec((256, 2)), _full_spec((1, 2))],
        out_specs=_row_spec(2),
        out_shape=jax.ShapeDtypeStruct((N, 2), _f32),
    )(h1, h2, h3, tx1, S[0], S[1], wcat,
      p["lab1_w"], l1b, lg, lb, p["lab2_w"], l2b)


def kernel(image_features, edge_index, non_image_features, params):
    p = params
    src = edge_index[0].astype(jnp.int32)
    dst = edge_index[1].astype(jnp.int32)

    ei, es, recon, sn2, c0a, u12, site_logits = _tc_premix(image_features, p)
    sn = sn2.reshape(N)
    pd, q1, q2 = _tc_parser(non_image_features, p)

    ew, deg = _sc_ew_deg(es, src, dst, pd.reshape(E), q1.reshape(E),
                         q2.reshape(E), sn)

    P, norm = _sc_l_apply(u12, src, dst, 32, ew=ew, deg=deg)
    pre0, t = _tc_combine0(P, c0a)
    (Q,) = _sc_l_apply(t, src, dst, 16, norm=norm)
    h = _tc_h1(pre0, Q)

    hs = [h]
    for name in ("cheb1", "cheb2"):
        (R,) = _sc_l_apply(h, src, dst, 16, norm=norm)
        tx1 = _tc_addhalves(R)
        (S,) = _sc_l_apply(tx1, src, dst, 16, norm=norm)
        h = _tc_layer(h, tx1, S, p[name])
        hs.append(h)

    (R,) = _sc_l_apply(h, src, dst, 16, norm=norm)
    tx1 = _tc_addhalves(R)
    (S,) = _sc_l_apply(tx1, src, dst, 16, norm=norm)
    label_logits = _tc_final(hs[0], hs[1], h, tx1, S, p["cheb3"], p)

    return (label_logits, site_logits, es, recon)


# speedup vs baseline: 13.1561x; 2.1023x over previous
import functools

import jax
import jax.numpy as jnp
from jax import lax
from jax.experimental import pallas as pl
from jax.experimental.pallas import tpu as pltpu
from jax.experimental.pallas import tpu_sc as plsc

N = 10000
E = 320000
NP = 10240
NC, NS = 2, 16
NW = NC * NS
EPT = E // NW
CH = 80
NCH = EPT // CH
SPAN = NP // NS
BN_S = float(1.0 / (1.0 + 1e-5) ** 0.5)

_f32 = jnp.float32


def _rsqrt16(x):
    i = plsc.bitcast(x, jnp.int32)
    i = jnp.full((16,), 0x5F3759DF, jnp.int32) - lax.shift_right_logical(i, 1)
    y = plsc.bitcast(i, _f32)
    for _ in range(4):
        y = y * (1.5 - 0.5 * x * y * y)
    return y


def _zero16():
    return jnp.zeros((16,), _f32)


def _sc_ew_deg(es, src3, dst3, pd3, q13, q23, sn):
    mesh = plsc.VectorSubcoreMesh(core_axis_name="c", subcore_axis_name="s")

    @functools.partial(
        pl.kernel,
        out_type=(jax.ShapeDtypeStruct((NW, NCH, CH), _f32),
                  jax.ShapeDtypeStruct((NC, NP), _f32)),
        mesh=mesh,
        compiler_params=pltpu.CompilerParams(needs_layout_passes=False, use_tc_tiling_on_sc=False),
        scratch_types=[
            pltpu.VMEM((N,), _f32),
            pltpu.VMEM((NCH, CH), jnp.int32),
            pltpu.VMEM((NCH, CH), jnp.int32),
            pltpu.VMEM((NCH, CH), _f32),
            pltpu.VMEM((NCH, CH), _f32),
            pltpu.VMEM((NCH, CH), _f32),
            pltpu.VMEM((NCH, CH), _f32),
            pltpu.VMEM((CH, 128), _f32),
            pltpu.VMEM((CH, 128), _f32),
            pltpu.VMEM((CH, 128), _f32),
            pltpu.VMEM((CH, 128), _f32),
            pltpu.VMEM((CH * 16,), _f32),
            pltpu.VMEM((SPAN,), _f32),
            pltpu.VMEM_SHARED((NP,), _f32),
            pltpu.SemaphoreType.DMA,
            pltpu.SemaphoreType.DMA,
        ],
    )
    def k(es_h, src_h, dst_h, pd_h, q1_h, q2_h, sn_h, ew_h, deg_h,
          sn_v, src_a, dst_a, pd_a, q1_a, q2_a, ew_a,
          e1a, e2a, e1b, e2b, prs, zv, deg_sp, semA, semB):
        cid = lax.axis_index("c")
        sid = lax.axis_index("s")
        wid = sid * NC + cid
        iota = lax.iota(jnp.int32, 16)

        pltpu.sync_copy(src_h.at[wid], src_a)
        pltpu.sync_copy(dst_h.at[wid], dst_a)
        pltpu.sync_copy(pd_h.at[wid], pd_a)
        pltpu.sync_copy(q1_h.at[wid], q1_a)
        pltpu.sync_copy(q2_h.at[wid], q2_a)
        pltpu.sync_copy(sn_h, sn_v)

        def issue(kk, e1x, e2x, semx):
            c1 = pltpu.async_copy(es_h.at[src_a.at[kk]], e1x, semx)
            c2 = pltpu.async_copy(es_h.at[dst_a.at[kk]], e2x, semx)
            return c1, c2

        p0 = issue(0, e1a, e2a, semA)

        def zb(i, c):
            zv[pl.ds(i * 16, 16)] = _zero16()
            return c
        lax.fori_loop(0, SPAN // 16, zb, 0)
        pltpu.sync_copy(zv, deg_sp.at[pl.ds(sid * SPAN, SPAN)])
        plsc.subcore_barrier()

        def process(kk, e1x, e2x):
            def edot(e, cc):
                acc = e1x[e, pl.ds(0, 16)] * e2x[e, pl.ds(0, 16)]
                for t in range(1, 8):
                    acc = acc + (e1x[e, pl.ds(t * 16, 16)]
                                 * e2x[e, pl.ds(t * 16, 16)])
                prs[pl.ds(e * 16, 16)] = acc
                return cc
            lax.fori_loop(0, CH, edot, 0, unroll=4)

            def group(g, cc):
                rows = (g * 16 + iota) * 16
                ed = plsc.load_gather(prs, [rows])
                for j in range(1, 16):
                    ed = ed + plsc.load_gather(prs, [rows + j])
                s16 = src_a[kk, pl.ds(g * 16, 16)]
                d16 = dst_a[kk, pl.ds(g * 16, 16)]
                sns = plsc.load_gather(sn_v, [s16])
                snd = plsc.load_gather(sn_v, [d16])
                num = pd_a[kk, pl.ds(g * 16, 16)] + ed
                den2 = jnp.maximum(
                    (q1_a[kk, pl.ds(g * 16, 16)] + sns)
                    * (q2_a[kk, pl.ds(g * 16, 16)] + snd), 1e-16)
                ew_a[kk, pl.ds(g * 16, 16)] = (
                    (num * _rsqrt16(den2) + 1.0) * 0.5)
                return cc
            lax.fori_loop(0, CH // 16, group, 0)

            pltpu.sync_copy(ew_a.at[kk], deg_sp.at[src_a.at[kk]], add=True)

        def pair(p, c):
            k0 = 2 * p
            qb = issue(k0 + 1, e1b, e2b, semB)
            p0[0].wait()
            p0[1].wait()
            process(k0, e1a, e2a)
            pa = issue(k0 + 2, e1a, e2a, semA)
            qb[0].wait()
            qb[1].wait()
            process(k0 + 1, e1b, e2b)
            return c
        lax.fori_loop(0, (NCH - 1) // 2, pair, 0)
        p0[0].wait()
        p0[1].wait()
        process(NCH - 1, e1a, e2a)

        pltpu.sync_copy(ew_a, ew_h.at[wid])
        plsc.subcore_barrier()
        pltpu.sync_copy(deg_sp.at[pl.ds(sid * SPAN, SPAN)],
                        deg_h.at[cid, pl.ds(sid * SPAN, SPAN)])

    return k(es, src3, dst3, pd3, q13, q23, sn)


def _sc_l_apply(x, src3, dst3, F, norm3=None, ew3=None, deg=None):
    mesh = plsc.VectorSubcoreMesh(core_axis_name="c", subcore_axis_name="s")
    with_norm = norm3 is None
    out_type = [jax.ShapeDtypeStruct((NC, NP, F), _f32)]
    if with_norm:
        out_type.append(jax.ShapeDtypeStruct((NW, NCH, CH), _f32))
    scratch = [
        pltpu.VMEM((NP,), _f32),
        pltpu.VMEM((NP,), _f32),
        pltpu.VMEM((NCH, CH), jnp.int32),
        pltpu.VMEM((NCH, CH), jnp.int32),
        pltpu.VMEM((NCH, CH), _f32),
        pltpu.VMEM((NCH, CH), _f32),
        pltpu.VMEM((CH, F), _f32),
        pltpu.VMEM((CH, F), _f32),
        pltpu.VMEM((SPAN, F), _f32),
        pltpu.VMEM_SHARED((NP, F), _f32),
        pltpu.SemaphoreType.DMA,
        pltpu.SemaphoreType.DMA,
    ]

    def body(*refs):
        if with_norm:
            (x_h, src_h, dst_h, ew_h, deg_h, acc_h, norm_h,
             dinv_v, tmp_v, src_a, dst_a, ew_a, nm_a, rva, rvb, zv,
             acc_sp, semA, semB) = refs
        else:
            (x_h, src_h, dst_h, norm_h, acc_h,
             dinv_v, tmp_v, src_a, dst_a, ew_a, nm_a, rva, rvb, zv,
             acc_sp, semA, semB) = refs
        cid = lax.axis_index("c")
        sid = lax.axis_index("s")
        wid = sid * NC + cid
        iota = lax.iota(jnp.int32, 16)

        pltpu.sync_copy(src_h.at[wid], src_a)
        pltpu.sync_copy(dst_h.at[wid], dst_a)

        def issue(kk, rvx, semx):
            return pltpu.async_copy(x_h.at[src_a.at[kk]], rvx, semx)

        p0 = issue(0, rva, semA)

        if with_norm:
            pltpu.sync_copy(ew_h.at[wid], ew_a)
            pltpu.sync_copy(deg_h.at[0], dinv_v)
            pltpu.sync_copy(deg_h.at[1], tmp_v)

            def db(i, c):
                d = dinv_v[pl.ds(i * 16, 16)] + tmp_v[pl.ds(i * 16, 16)]
                r = _rsqrt16(jnp.maximum(d, 1e-30))
                dinv_v[pl.ds(i * 16, 16)] = jnp.where(d > 0.0, r, 0.0)
                return c
            lax.fori_loop(0, NP // 16, db, 0, unroll=2)

            def nrm(i, c):
                kk = i // (CH // 16)
                gg = i % (CH // 16)
                s16 = src_a[kk, pl.ds(gg * 16, 16)]
                d16 = dst_a[kk, pl.ds(gg * 16, 16)]
                nv = (-ew_a[kk, pl.ds(gg * 16, 16)]
                      * plsc.load_gather(dinv_v, [s16])
                      * plsc.load_gather(dinv_v, [d16]))
                nm_a[kk, pl.ds(gg * 16, 16)] = nv
                return c
            lax.fori_loop(0, NCH * (CH // 16), nrm, 0, unroll=2)
            pltpu.sync_copy(nm_a, norm_h.at[wid])
        else:
            pltpu.sync_copy(norm_h.at[wid], nm_a)

        def zb(i, c):
            for t in range(F // 16):
                zv[i, pl.ds(t * 16, 16)] = _zero16()
            return c
        lax.fori_loop(0, SPAN, zb, 0, unroll=4)
        pltpu.sync_copy(zv, acc_sp.at[pl.ds(sid * SPAN, SPAN)])
        plsc.subcore_barrier()

        def process(kk, rvx):
            def scale(e, cc):
                nv = jnp.full((16,), nm_a[kk, pl.ds(e, 16)][0])
                for t in range(F // 16):
                    rvx[e, pl.ds(t * 16, 16)] = (
                        rvx[e, pl.ds(t * 16, 16)] * nv)
                return cc
            lax.fori_loop(0, CH, scale, 0, unroll=4)
            pltpu.sync_copy(rvx, acc_sp.at[dst_a.at[kk]], add=True)

        def pair(p, c):
            k0 = 2 * p
            qb = issue(k0 + 1, rvb, semB)
            p0.wait()
            process(k0, rva)
            issue(k0 + 2, rva, semA)
            qb.wait()
            process(k0 + 1, rvb)
            return c
        lax.fori_loop(0, (NCH - 1) // 2, pair, 0)
        p0.wait()
        process(NCH - 1, rva)

        plsc.subcore_barrier()
        pltpu.sync_copy(acc_sp.at[pl.ds(sid * SPAN, SPAN)],
                        acc_h.at[cid, pl.ds(sid * SPAN, SPAN)])

    kfn = functools.partial(
        pl.kernel, out_type=tuple(out_type), mesh=mesh,
        scratch_types=scratch,
        compiler_params=pltpu.CompilerParams(needs_layout_passes=False, use_tc_tiling_on_sc=False))(body)
    if with_norm:
        return kfn(x, src3, dst3, ew3, deg)
    return kfn(x, src3, dst3, norm3)


_BN_NODE = 2000
_GRID_N = N // _BN_NODE


def _row_spec(width):
    return pl.BlockSpec((_BN_NODE, width), lambda i: (i, 0))


def _full_spec(shape):
    nd = len(shape)
    return pl.BlockSpec(shape, lambda i: (0,) * nd)


def _tc_premix(x, p):
    eib = p["EI_b"].reshape(1, 128)
    esb = p["ES_b"].reshape(1, 128)
    deb = p["DE_b"].reshape(1, 128)
    cwa = p["cheb0"][0] - p["cheb0"][2]
    cw12 = jnp.concatenate([p["cheb0"][1], p["cheb0"][2]], axis=1)
    s1b = p["site1_b"].reshape(1, 256)
    sg = (p["site_bn_g"] * BN_S).reshape(1, 256)
    sb = p["site_bn_b"].reshape(1, 256)
    s2b = p["site2_b"].reshape(1, 20)

    def body(x_r, eiw, eib_r, esw, esb_r, dew, deb_r, cwa_r, cw12_r,
             s1w, s1b_r, sg_r, sb_r, s2w, s2b_r,
             ei_o, es_o, rec_o, sn_o, c0a_o, u12_o, site_o):
        xx = x_r[...]
        ei = jnp.dot(xx, eiw[...], preferred_element_type=_f32) + eib_r[...]
        es = jnp.dot(xx, esw[...], preferred_element_type=_f32) + esb_r[...]
        ei_o[...] = ei
        es_o[...] = es
        dw = dew[...]
        rec_o[...] = (jnp.dot(ei, dw[:128, :], preferred_element_type=_f32)
                      + jnp.dot(es, dw[128:, :], preferred_element_type=_f32)
                      + deb_r[...])
        sn_o[...] = jnp.sum(es * es, axis=1, keepdims=True)
        c0a_o[...] = jnp.dot(ei, cwa_r[...], preferred_element_type=_f32)
        u12_o[...] = jnp.dot(ei, cw12_r[...], preferred_element_type=_f32)
        s = jnp.maximum(
            jnp.dot(ei, s1w[...], preferred_element_type=_f32) + s1b_r[...],
            0.0)
        s = s * sg_r[...] + sb_r[...]
        site_o[...] = jnp.dot(s, s2w[...], preferred_element_type=_f32) + s2b_r[...]

    return pl.pallas_call(
        body,
        grid=(_GRID_N,),
        in_specs=[
            _row_spec(128),
            _full_spec((128, 128)), _full_spec((1, 128)),
            _full_spec((128, 128)), _full_spec((1, 128)),
            _full_spec((256, 128)), _full_spec((1, 128)),
            _full_spec((128, 16)), _full_spec((128, 32)),
            _full_spec((128, 256)), _full_spec((1, 256)),
            _full_spec((1, 256)), _full_spec((1, 256)),
            _full_spec((256, 20)), _full_spec((1, 20)),
        ],
        out_specs=[
            _row_spec(128), _row_spec(128), _row_spec(128), _row_spec(1),
            _row_spec(16), _row_spec(32), _row_spec(20),
        ],
        out_shape=[
            jax.ShapeDtypeStruct((N, 128), _f32),
            jax.ShapeDtypeStruct((N, 128), _f32),
            jax.ShapeDtypeStruct((N, 128), _f32),
            jax.ShapeDtypeStruct((N, 1), _f32),
            jax.ShapeDtypeStruct((N, 16), _f32),
            jax.ShapeDtypeStruct((N, 32), _f32),
            jax.ShapeDtypeStruct((N, 20), _f32),
        ],
    )(x, p["EI_w"], eib, p["ES_w"], esb, p["DE_w"], deb, cwa, cw12,
      p["site1_w"], s1b, sg, sb, p["site2_w"], s2b)


_BE = 2000
_GRID_E = E // _BE


def _tc_parser(nf, p):
    zpad = jnp.zeros((E, 5), _f32)
    nfp = jnp.concatenate([nf[:, :3], zpad, nf[:, 3:], zpad], axis=1)
    w1 = jnp.concatenate([p["pewe_l1_w"], jnp.zeros((5, 128), _f32)], axis=0)
    b1 = p["pewe_l1_b"].reshape(1, 128)
    g1 = (p["pewe_bn_g"] * BN_S).reshape(1, 128)
    bb1 = p["pewe_bn_b"].reshape(1, 128)
    b2 = p["pewe_l2_b"].reshape(1, 128)

    def body(nf_r, w1_r, b1_r, g1_r, bb1_r, w2_r, b2_r, pd_o, q1_o, q2_o):
        xf = nf_r[...]

        def pars(xx):
            h = jnp.maximum(
                jnp.dot(xx, w1_r[...], preferred_element_type=_f32)
                + b1_r[...], 0.0)
            h = h * g1_r[...] + bb1_r[...]
            return jnp.dot(h, w2_r[...], preferred_element_type=_f32) + b2_r[...]

        p1 = pars(xf[:, 0:8])
        p2 = pars(xf[:, 8:16])
        pd_o[...] = jnp.sum(p1 * p2, axis=1, keepdims=True)
        q1_o[...] = jnp.sum(p1 * p1, axis=1, keepdims=True)
        q2_o[...] = jnp.sum(p2 * p2, axis=1, keepdims=True)

    espec = pl.BlockSpec((_BE, 16), lambda i: (i, 0))
    ospec = pl.BlockSpec((_BE, 1), lambda i: (i, 0))
    return pl.pallas_call(
        body,
        grid=(_GRID_E,),
        in_specs=[espec,
                  _full_spec((8, 128)), _full_spec((1, 128)),
                  _full_spec((1, 128)), _full_spec((1, 128)),
                  _full_spec((128, 128)), _full_spec((1, 128))],
        out_specs=[ospec, ospec, ospec],
        out_shape=[jax.ShapeDtypeStruct((E, 1), _f32)] * 3,
    )(nfp, w1, b1, g1, bb1, p["pewe_l2_w"], b2)


def _tc_combine0(P, c0a):
    def body(p0_r, p1_r, c0a_r, pre_o, t_o):
        s = p0_r[...] + p1_r[...]
        pre_o[...] = c0a_r[...] + s[:, :16]
        t_o[...] = s[:, 16:]

    return pl.pallas_call(
        body,
        grid=(_GRID_N,),
        in_specs=[_row_spec(32), _row_spec(32), _row_spec(16)],
        out_specs=[_row_spec(16), _row_spec(16)],
        out_shape=[jax.ShapeDtypeStruct((N, 16), _f32)] * 2,
    )(P[0], P[1], c0a)


def _tc_h1(pre0, Q):
    def body(pre_r, q0_r, q1_r, h_o):
        h_o[...] = jnp.maximum(
            pre_r[...] + 2.0 * (q0_r[...] + q1_r[...]), 0.0)

    return pl.pallas_call(
        body,
        grid=(_GRID_N,),
        in_specs=[_row_spec(16), _row_spec(16), _row_spec(16)],
        out_specs=_row_spec(16),
        out_shape=jax.ShapeDtypeStruct((N, 16), _f32),
    )(pre0, Q[0], Q[1])


def _tc_addhalves(R):
    def body(r0_r, r1_r, o_r):
        o_r[...] = r0_r[...] + r1_r[...]

    return pl.pallas_call(
        body,
        grid=(_GRID_N,),
        in_specs=[_row_spec(16), _row_spec(16)],
        out_specs=_row_spec(16),
        out_shape=jax.ShapeDtypeStruct((N, 16), _f32),
    )(R[0], R[1])


def _tc_layer(h, tx1, S, W):
    wcat = jnp.concatenate([W[0], W[1], W[2]], axis=0)

    def body(h_r, t_r, s0_r, s1_r, w_r, o_r):
        hh = h_r[...]
        tx2 = 2.0 * (s0_r[...] + s1_r[...]) - hh
        xcat = jnp.concatenate([hh, t_r[...], tx2], axis=1)
        o_r[...] = jnp.maximum(
            jnp.dot(xcat, w_r[...], preferred_element_type=_f32), 0.0)

    return pl.pallas_call(
        body,
        grid=(_GRID_N,),
        in_specs=[_row_spec(16), _row_spec(16), _row_spec(16), _row_spec(16),
                  _full_spec((48, 16))],
        out_specs=_row_spec(16),
        out_shape=jax.ShapeDtypeStruct((N, 16), _f32),
    )(h, tx1, S[0], S[1], wcat)


def _tc_final(h1, h2, h3, tx1, S, W, p):
    wcat = jnp.concatenate([W[0], W[1], W[2]], axis=0)
    l1b = p["lab1_b"].reshape(1, 256)
    lg = (p["lab_bn_g"] * BN_S).reshape(1, 256)
    lb = p["lab_bn_b"].reshape(1, 256)
    l2b = p["lab2_b"].reshape(1, 2)

    def body(h1_r, h2_r, h3_r, t_r, s0_r, s1_r, w_r,
             l1w_r, l1b_r, lg_r, lb_r, l2w_r, l2b_r, o_r):
        h3v = h3_r[...]
        tx2 = 2.0 * (s0_r[...] + s1_r[...]) - h3v
        xcat = jnp.concatenate([h3v, t_r[...], tx2], axis=1)
        h4 = jnp.maximum(
            jnp.dot(xcat, w_r[...], preferred_element_type=_f32), 0.0)
        jk = jnp.concatenate([h1_r[...], h2_r[...], h3v, h4], axis=1)
        z = jnp.maximum(
            jnp.dot(jk, l1w_r[...], preferred_element_type=_f32)
            + l1b_r[...], 0.0)
        z = z * lg_r[...] + lb_r[...]
        o_r[...] = jnp.dot(z, l2w_r[...], preferred_element_type=_f32) + l2b_r[...]

    return pl.pallas_call(
        body,
        grid=(_GRID_N,),
        in_specs=[_row_spec(16)] * 6 + [
            _full_spec((48, 16)),
            _full_spec((64, 256)), _full_spec((1, 256)),
            _full_spec((1, 256)), _full_spec((1, 256)),
            _full_spec((256, 2)), _full_spec((1, 2))],
        out_specs=_row_spec(2),
        out_shape=jax.ShapeDtypeStruct((N, 2), _f32),
    )(h1, h2, h3, tx1, S[0], S[1], wcat,
      p["lab1_w"], l1b, lg, lb, p["lab2_w"], l2b)


def kernel(image_features, edge_index, non_image_features, params):
    p = params
    src3 = edge_index[0].astype(jnp.int32).reshape(NW, NCH, CH)
    dst3 = edge_index[1].astype(jnp.int32).reshape(NW, NCH, CH)

    ei, es, recon, sn2, c0a, u12, site_logits = _tc_premix(image_features, p)
    sn = sn2.reshape(N)
    pd, q1, q2 = _tc_parser(non_image_features, p)

    ew3, deg = _sc_ew_deg(es, src3, dst3, pd.reshape(NW, NCH, CH),
                          q1.reshape(NW, NCH, CH), q2.reshape(NW, NCH, CH),
                          sn)

    P, norm3 = _sc_l_apply(u12, src3, dst3, 32, ew3=ew3, deg=deg)
    pre0, t = _tc_combine0(P, c0a)
    (Q,) = _sc_l_apply(t, src3, dst3, 16, norm3=norm3)
    h = _tc_h1(pre0, Q)

    hs = [h]
    for name in ("cheb1", "cheb2"):
        (R,) = _sc_l_apply(h, src3, dst3, 16, norm3=norm3)
        tx1 = _tc_addhalves(R)
        (S,) = _sc_l_apply(tx1, src3, dst3, 16, norm3=norm3)
        h = _tc_layer(h, tx1, S, p[name])
        hs.append(h)

    (R,) = _sc_l_apply(h, src3, dst3, 16, norm3=norm3)
    tx1 = _tc_addhalves(R)
    (S,) = _sc_l_apply(tx1, src3, dst3, 16, norm3=norm3)
    label_logits = _tc_final(hs[0], hs[1], h, tx1, S, p["cheb3"], p)

    return (label_logits, site_logits, es, recon)
